# baseline (speedup 1.0000x reference)
import jax
import jax.numpy as jnp
from jax import lax
from jax.experimental import pallas as pl
from jax.experimental.pallas import tpu as pltpu
from jax.experimental.pallas import tpu_sc as plsc

B = 65536
L = 16
NC = 2
NS = 16
NW = NC * NS
STRIPE = B // NS
BLK = B // NW
CH2 = BLK // L
FBUF = BLK + L


def _nnp(sv):
    return jnp.right_shift(sv + 15, 4) - jnp.right_shift(sv + 14, 4)


def _body(seq_hbm, last_hbm, free_hbm, out_hbm,
          seq1_v, seq2_v, last_v, free_v, out_v, stage_v, tots_v, tot_hbm,
          sem_seq2, sem_last):
    c = lax.axis_index("c")
    s = lax.axis_index("s")
    k = c * NS + s
    base = k * BLK

    cp_seq2 = pltpu.async_copy(seq_hbm.at[pl.ds(base, BLK)], seq2_v, sem_seq2)
    cp_last = pltpu.async_copy(last_hbm.at[pl.ds(base, BLK)], last_v, sem_last)

    pltpu.sync_copy(seq_hbm.at[pl.ds(s * STRIPE, STRIPE)], seq1_v)

    for b in range(2):
        @plsc.parallel_loop(0, CH2, unroll=8,
                            carry=jnp.zeros((L,), jnp.int32))
        def p1(i, acc, b=b):
            sv = seq1_v[pl.ds(b * BLK + i * L, L)]
            return acc + _nnp(sv)
        stage_v[b] = jnp.full((L,), jnp.sum(p1), jnp.int32)

    pltpu.sync_copy(stage_v, tot_hbm.at[c].at[pl.ds(2 * s, 2)])
    plsc.subcore_barrier()
    pltpu.sync_copy(tot_hbm.at[c], tots_v)

    off = jnp.zeros((L,), jnp.int32)
    for j in range(NW):
        off = jnp.where(j < k, off + tots_v[j], off)
    off_s = jnp.max(off)

    start = pl.multiple_of(
        jnp.minimum(jnp.bitwise_and(off_s, -L), B - FBUF), L)
    pltpu.sync_copy(free_hbm.at[pl.ds(start, FBUF)], free_v)
    adj = off_s - start
    cp_seq2.wait()
    cp_last.wait()

    adj_v = off - jnp.full((L,), start, jnp.int32)

    @plsc.parallel_loop(0, CH2, unroll=4,
                        carry=jnp.zeros((L,), jnp.int32))
    def p2(i, carry):
        sv = seq2_v[pl.ds(i * L, L)]
        nnp = _nnp(sv)
        mask = nnp != 0
        inc = plsc.cumsum(nnp)
        excl = inc - nnp + carry
        idx = jnp.minimum(excl + adj_v, FBUF - 1)
        page = plsc.load_gather(free_v, [idx]) * L
        ll = last_v[pl.ds(i * L, L)]
        out_v[pl.ds(i * L, L)] = jnp.where(mask, page, ll + 1)
        return carry + plsc.all_reduce_population_count(mask)

    pltpu.sync_copy(out_v, out_hbm.at[pl.ds(base, BLK)])


def kernel(seq_lens, last_loc, free_page):
    run = pl.kernel(
        _body,
        out_type=jax.ShapeDtypeStruct((B,), jnp.int32),
        mesh=plsc.VectorSubcoreMesh(core_axis_name="c", subcore_axis_name="s"),
        compiler_params=pltpu.CompilerParams(needs_layout_passes=False),
        scratch_types=[
            pltpu.VMEM((STRIPE,), jnp.int32),
            pltpu.VMEM((BLK,), jnp.int32),
            pltpu.VMEM((BLK,), jnp.int32),
            pltpu.VMEM((FBUF,), jnp.int32),
            pltpu.VMEM((BLK,), jnp.int32),
            pltpu.VMEM((2, L), jnp.int32),
            pltpu.VMEM((NW, L), jnp.int32),
            pltpu.MemorySpace.HBM((NC, NW, L), jnp.int32),
            pltpu.SemaphoreType.DMA,
            pltpu.SemaphoreType.DMA,
        ],
    )
    return run(seq_lens.astype(jnp.int32),
               last_loc.astype(jnp.int32),
               free_page.astype(jnp.int32))


# speedup vs baseline: 1.0838x; 1.0838x over previous
import jax
import jax.numpy as jnp
from jax import lax
from jax.experimental import pallas as pl
from jax.experimental.pallas import tpu as pltpu
from jax.experimental.pallas import tpu_sc as plsc

B = 65536
L = 16
NS = 16
BLK = B // NS
CH = BLK // L
FAST = 8192
SBUF = BLK + L


def _nnp(sv):
    return jnp.right_shift(sv + 15, 4) - jnp.right_shift(sv + 14, 4)


def _body(seq_hbm, last_hbm, free_hbm, out_hbm,
          seq_v, last_v, free_v, out_v, stage_v, tots_v, tot_hbm,
          sem_last, sem_free):
    s = lax.axis_index("s")
    base = s * BLK

    cp_last = pltpu.async_copy(last_hbm.at[pl.ds(base, BLK)], last_v, sem_last)
    cp_free = pltpu.async_copy(free_hbm.at[pl.ds(0, FAST)], free_v, sem_free)

    pltpu.sync_copy(seq_hbm.at[pl.ds(base, BLK)], seq_v)

    @plsc.parallel_loop(0, CH, unroll=8, carry=jnp.zeros((L,), jnp.int32))
    def p1(i, acc):
        return acc + _nnp(seq_v[pl.ds(i * L, L)])

    tot_own = jnp.sum(p1)
    stage_v[...] = jnp.full((L,), tot_own, jnp.int32)
    pltpu.sync_copy(stage_v, tot_hbm.at[s])
    plsc.subcore_barrier()
    pltpu.sync_copy(tot_hbm, tots_v)

    off = jnp.zeros((L,), jnp.int32)
    for j in range(NS):
        off = jnp.where(j < s, off + tots_v[j], off)
    off_s = jnp.max(off)

    cp_free.wait()
    slow = off_s + tot_own > FAST
    start = pl.multiple_of(
        jnp.minimum(jnp.bitwise_and(off_s, -L), B - SBUF), L)

    @pl.when(slow)
    def _():
        pltpu.sync_copy(free_hbm.at[pl.ds(start, SBUF)],
                        free_v.at[pl.ds(0, SBUF)])

    base_v = jnp.where(slow, off - jnp.full((L,), start, jnp.int32), off)
    cp_last.wait()

    @plsc.parallel_loop(0, CH, unroll=4, carry=jnp.zeros((L,), jnp.int32))
    def p2(i, carry):
        sv = seq_v[pl.ds(i * L, L)]
        nnp = _nnp(sv)
        mask = nnp != 0
        inc = plsc.cumsum(nnp)
        excl = inc - nnp + carry
        idx = jnp.minimum(excl + base_v, FAST - 1)
        page = plsc.load_gather(free_v, [idx]) * L
        ll = last_v[pl.ds(i * L, L)]
        out_v[pl.ds(i * L, L)] = jnp.where(mask, page, ll + 1)
        return carry + plsc.all_reduce_population_count(mask)

    pltpu.sync_copy(out_v, out_hbm.at[pl.ds(base, BLK)])


def kernel(seq_lens, last_loc, free_page):
    run = pl.kernel(
        _body,
        out_type=jax.ShapeDtypeStruct((B,), jnp.int32),
        mesh=plsc.VectorSubcoreMesh(core_axis_name="c", subcore_axis_name="s",
                                    num_cores=1),
        compiler_params=pltpu.CompilerParams(needs_layout_passes=False),
        scratch_types=[
            pltpu.VMEM((BLK,), jnp.int32),
            pltpu.VMEM((BLK,), jnp.int32),
            pltpu.VMEM((FAST,), jnp.int32),
            pltpu.VMEM((BLK,), jnp.int32),
            pltpu.VMEM((L,), jnp.int32),
            pltpu.VMEM((NS, L), jnp.int32),
            pltpu.MemorySpace.HBM((NS, L), jnp.int32),
            pltpu.SemaphoreType.DMA,
            pltpu.SemaphoreType.DMA,
        ],
    )
    return run(seq_lens.astype(jnp.int32),
               last_loc.astype(jnp.int32),
               free_page.astype(jnp.int32))


# speedup vs baseline: 1.1427x; 1.0543x over previous
import jax
import jax.numpy as jnp
from jax import lax
from jax.experimental import pallas as pl
from jax.experimental.pallas import tpu as pltpu
from jax.experimental.pallas import tpu_sc as plsc

B = 65536
L = 16
NS = 16
BLK = B // NS
CH = BLK // L
FAST = 8192
SBUF = BLK + L
PAD = 32


def _nnp(sv):
    return jnp.right_shift(sv + 15, 4) - jnp.right_shift(sv + 14, 4)


def _body(seq_hbm, last_hbm, free_hbm, out_hbm,
          seq_v, last_v, free_v, out_v, stage_v, tots_v, tot_spm,
          sem_last, sem_free):
    s = lax.axis_index("s")
    base = s * BLK

    cp_last = pltpu.async_copy(last_hbm.at[pl.ds(base, BLK)], last_v, sem_last)
    cp_free = pltpu.async_copy(free_hbm.at[pl.ds(0, FAST)], free_v, sem_free)

    pltpu.sync_copy(seq_hbm.at[pl.ds(base, BLK)], seq_v)

    @plsc.parallel_loop(0, CH, unroll=8, carry=jnp.zeros((L,), jnp.int32))
    def p1(i, acc):
        return acc + _nnp(seq_v[pl.ds(i * L, L)])

    tot_own = jnp.sum(p1)
    stage_v[...] = jnp.full((L,), tot_own, jnp.int32)
    pltpu.sync_copy(stage_v, tot_spm.at[PAD + s])
    plsc.subcore_barrier()
    pltpu.sync_copy(tot_spm.at[pl.ds(PAD, NS)], tots_v)

    off = jnp.zeros((L,), jnp.int32)
    for j in range(NS):
        off = jnp.where(j < s, off + tots_v[j], off)
    off_s = jnp.max(off)

    cp_free.wait()
    slow = off_s + tot_own > FAST
    start = pl.multiple_of(
        jnp.minimum(jnp.bitwise_and(off_s, -L), B - SBUF), L)

    @pl.when(slow)
    def _():
        pltpu.sync_copy(free_hbm.at[pl.ds(start, SBUF)],
                        free_v.at[pl.ds(0, SBUF)])

    base_v = jnp.where(slow, off - jnp.full((L,), start, jnp.int32), off)
    cp_last.wait()

    @plsc.parallel_loop(0, CH, unroll=4, carry=jnp.zeros((L,), jnp.int32))
    def p2(i, carry):
        sv = seq_v[pl.ds(i * L, L)]
        nnp = _nnp(sv)
        mask = nnp != 0
        inc = plsc.cumsum(nnp)
        excl = inc - nnp + carry
        idx = jnp.minimum(excl + base_v, FAST - 1)
        page = plsc.load_gather(free_v, [idx]) * L
        ll = last_v[pl.ds(i * L, L)]
        out_v[pl.ds(i * L, L)] = jnp.where(mask, page, ll + 1)
        return carry + plsc.all_reduce_population_count(mask)

    pltpu.sync_copy(out_v, out_hbm.at[pl.ds(base, BLK)])


def kernel(seq_lens, last_loc, free_page):
    run = pl.kernel(
        _body,
        out_type=jax.ShapeDtypeStruct((B,), jnp.int32),
        mesh=plsc.VectorSubcoreMesh(core_axis_name="c", subcore_axis_name="s",
                                    num_cores=1),
        compiler_params=pltpu.CompilerParams(needs_layout_passes=False),
        scratch_types=[
            pltpu.VMEM((BLK,), jnp.int32),
            pltpu.VMEM((BLK,), jnp.int32),
            pltpu.VMEM((FAST,), jnp.int32),
            pltpu.VMEM((BLK,), jnp.int32),
            pltpu.VMEM((L,), jnp.int32),
            pltpu.VMEM((NS, L), jnp.int32),
            pltpu.VMEM_SHARED((PAD + NS, L), jnp.int32),
            pltpu.SemaphoreType.DMA,
            pltpu.SemaphoreType.DMA,
        ],
    )
    return run(seq_lens.astype(jnp.int32),
               last_loc.astype(jnp.int32),
               free_page.astype(jnp.int32))


# speedup vs baseline: 1.1462x; 1.0031x over previous
import jax
import jax.numpy as jnp
from jax import lax
from jax.experimental import pallas as pl
from jax.experimental.pallas import tpu as pltpu
from jax.experimental.pallas import tpu_sc as plsc

B = 65536
L = 16
NS = 16
BLK = B // NS
CH = BLK // L
FAST = 8192
SBUF = BLK + L
PAD = 32


def _cross(sv):
    return jnp.bitwise_and(sv, 15) == 1


def _body(seq_hbm, last_hbm, free_hbm, out_hbm,
          seq_v, last_v, free_v, out_v, stage_v, tots_v, tot_spm,
          sem_last, sem_free):
    s = lax.axis_index("s")
    base = s * BLK

    cp_last = pltpu.async_copy(last_hbm.at[pl.ds(base, BLK)], last_v, sem_last)
    cp_free = pltpu.async_copy(free_hbm.at[pl.ds(0, FAST)], free_v, sem_free)

    pltpu.sync_copy(seq_hbm.at[pl.ds(base, BLK)], seq_v)

    one = jnp.full((L,), 1, jnp.int32)
    zero = jnp.zeros((L,), jnp.int32)

    @plsc.parallel_loop(0, CH, unroll=16, carry=jnp.zeros((L,), jnp.int32))
    def p1(i, acc):
        return acc + jnp.where(_cross(seq_v[pl.ds(i * L, L)]), one, zero)

    tot_own = jnp.sum(p1)
    stage_v[...] = jnp.full((L,), tot_own, jnp.int32)
    pltpu.sync_copy(stage_v, tot_spm.at[PAD + s])
    plsc.subcore_barrier()
    pltpu.sync_copy(tot_spm.at[pl.ds(PAD, NS)], tots_v)

    off = jnp.zeros((L,), jnp.int32)
    for j in range(NS):
        off = jnp.where(j < s, off + tots_v[j], off)
    off_s = jnp.max(off)

    cp_free.wait()
    slow = off_s + tot_own > FAST
    start = pl.multiple_of(
        jnp.minimum(jnp.bitwise_and(off_s, -L), B - SBUF), L)

    @pl.when(slow)
    def _():
        pltpu.sync_copy(free_hbm.at[pl.ds(start, SBUF)],
                        free_v.at[pl.ds(0, SBUF)])

    base_v = jnp.where(slow, off - jnp.full((L,), start, jnp.int32), off)
    cp_last.wait()

    @plsc.parallel_loop(0, CH, unroll=8, carry=jnp.zeros((L,), jnp.int32))
    def p2(i, carry):
        sv = seq_v[pl.ds(i * L, L)]
        mask = _cross(sv)
        nnp = jnp.where(mask, one, zero)
        inc = plsc.cumsum(nnp)
        excl = inc - nnp + carry
        idx = jnp.minimum(excl + base_v, FAST - 1)
        page = plsc.load_gather(free_v, [idx]) * L
        ll = last_v[pl.ds(i * L, L)]
        out_v[pl.ds(i * L, L)] = jnp.where(mask, page, ll + 1)
        return carry + plsc.all_reduce_population_count(mask)

    pltpu.sync_copy(out_v, out_hbm.at[pl.ds(base, BLK)])


def kernel(seq_lens, last_loc, free_page):
    run = pl.kernel(
        _body,
        out_type=jax.ShapeDtypeStruct((B,), jnp.int32),
        mesh=plsc.VectorSubcoreMesh(core_axis_name="c", subcore_axis_name="s",
                                    num_cores=1),
        compiler_params=pltpu.CompilerParams(needs_layout_passes=False),
        scratch_types=[
            pltpu.VMEM((BLK,), jnp.int32),
            pltpu.VMEM((BLK,), jnp.int32),
            pltpu.VMEM((FAST,), jnp.int32),
            pltpu.VMEM((BLK,), jnp.int32),
            pltpu.VMEM((L,), jnp.int32),
            pltpu.VMEM((NS, L), jnp.int32),
            pltpu.VMEM_SHARED((PAD + NS, L), jnp.int32),
            pltpu.SemaphoreType.DMA,
            pltpu.SemaphoreType.DMA,
        ],
    )
    return run(seq_lens.astype(jnp.int32),
               last_loc.astype(jnp.int32),
               free_page.astype(jnp.int32))
